# baseline (speedup 1.0000x reference)
import functools

import jax
import jax.numpy as jnp
from jax import lax
from jax.experimental import pallas as pl
from jax.experimental.pallas import tpu as pltpu
from jax.experimental.pallas import tpu_sc as plsc

SEQ_LEN = 200
BATCH = 1024
DIM = 64
B = SEQ_LEN * BATCH
NC = 2
NS = 16
NW = NC * NS
BPW = B // NW
C = 128
NCHUNK = BPW // C

_mesh = plsc.VectorSubcoreMesh(core_axis_name="c", subcore_axis_name="s")


@functools.partial(
    pl.kernel,
    mesh=_mesh,
    compiler_params=pltpu.CompilerParams(use_tc_tiling_on_sc=False),
    out_type=jax.ShapeDtypeStruct((B, DIM), jnp.float32),
    scratch_types=[
        pltpu.VMEM((C,), jnp.int32),
        pltpu.VMEM((C, DIM), jnp.float32),
        pltpu.SemaphoreType.DMA,
    ],
)
def _gather(idx_hbm, table_hbm, out_hbm, idx_v, rows_v, sem):
    wid = lax.axis_index("s") * NC + lax.axis_index("c")
    base = wid * BPW

    def body(g, carry):
        off = base + g * C
        pltpu.sync_copy(idx_hbm.at[pl.ds(off, C)], idx_v)
        pltpu.async_copy(table_hbm.at[idx_v], rows_v, sem).wait()
        pltpu.sync_copy(rows_v, out_hbm.at[pl.ds(off, C)])
        return carry

    lax.fori_loop(0, NCHUNK, body, 0)


def kernel(source, W):
    idx = source.reshape(B)
    out = _gather(idx, W)
    return out.reshape(SEQ_LEN, BATCH, DIM)


# speedup vs baseline: 1.0757x; 1.0757x over previous
import functools

import jax
import jax.numpy as jnp
from jax import lax
from jax.experimental import pallas as pl
from jax.experimental.pallas import tpu as pltpu
from jax.experimental.pallas import tpu_sc as plsc

SEQ_LEN = 200
BATCH = 1024
DIM = 64
B = SEQ_LEN * BATCH
NC = 2
NS = 16
NW = NC * NS
BPW = B // NW
G = 128
NCHUNK = BPW // G
NBUF = 5
NSUPER = NCHUNK // NBUF

_mesh = plsc.VectorSubcoreMesh(core_axis_name="c", subcore_axis_name="s")


@functools.partial(
    pl.kernel,
    mesh=_mesh,
    compiler_params=pltpu.CompilerParams(use_tc_tiling_on_sc=False),
    out_type=jax.ShapeDtypeStruct((B, DIM), jnp.float32),
    scratch_types=[
        pltpu.VMEM((NCHUNK, G), jnp.int32),
        pltpu.VMEM((NBUF, G, DIM), jnp.float32),
    ]
    + [pltpu.SemaphoreType.DMA] * (2 * NBUF),
)
def _gather(idx_hbm, table_hbm, out_hbm, idx_v, rows_v, *sems):
    gsem = sems[:NBUF]
    wsem = sems[NBUF:]
    wid = lax.axis_index("s") * NC + lax.axis_index("c")
    chunk0 = wid * NCHUNK

    pltpu.sync_copy(idx_hbm.at[pl.ds(chunk0, NCHUNK)], idx_v)

    def fire_gather(g, b):
        pltpu.async_copy(table_hbm.at[idx_v.at[g]], rows_v.at[b], gsem[b])

    def wait_gather(b):
        pltpu.make_async_copy(
            table_hbm.at[idx_v.at[0]], rows_v.at[b], gsem[b]
        ).wait()

    def fire_wb(g, b):
        pltpu.async_copy(
            rows_v.at[b], out_hbm.at[pl.ds((chunk0 + g) * G, G)], wsem[b]
        )

    def wait_wb(b):
        pltpu.make_async_copy(
            rows_v.at[b], out_hbm.at[pl.ds(0, G)], wsem[b]
        ).wait()

    for b in range(NBUF):
        fire_gather(b, b)

    def body(s, carry):
        base = s * NBUF
        for b in range(NBUF):
            wait_gather(b)
            fire_wb(base + b, b)
        for b in range(NBUF):
            wait_wb(b)
            fire_gather(base + NBUF + b, b)
        return carry

    lax.fori_loop(0, NSUPER - 1, body, 0)

    last = (NSUPER - 1) * NBUF
    for b in range(NBUF):
        wait_gather(b)
        fire_wb(last + b, b)
    for b in range(NBUF):
        wait_wb(b)


def kernel(source, W):
    idx = source.reshape(NW * NCHUNK, G)
    out = _gather(idx, W)
    return out.reshape(SEQ_LEN, BATCH, DIM)
